# Initial kernel scaffold; baseline (speedup 1.0000x reference)
import jax
import jax.numpy as jnp
from jax.experimental import pallas as pl


def kernel(x, edge_index, W1, att_src1, att_dst1, bias1, W2, att_src2, att_dst2, bias2, ln_w, ln_b, Wi1, Wh1, bi1, bh1, Wi2, Wh2, bi2, bh2, fc_w, fc_b):
    raise NotImplementedError("write your pallas kernel here")


# speedup vs baseline: 38.2863x; 38.2863x over previous
import functools

import jax
import jax.numpy as jnp
from jax import lax
from jax.experimental import pallas as pl
from jax.experimental.pallas import tpu as pltpu
from jax.experimental.pallas import tpu_sc as plsc

B, N, T = 512, 32, 12
HID, HEADS, OUT = 64, 4, 12
NUM_NODES = B * N
E = 262144

NC, NS, L = 2, 16, 16
K = 128
STRIPE = NUM_NODES // NS


def _elu(x):
    return jnp.where(x > 0, x, jnp.exp(jnp.minimum(x, 0.0)) - 1.0)


def _stage_a_body(x_ref, w1_ref, asw_ref, adw_ref, h1_ref, a1_ref):
    xb = x_ref[...]
    h = jnp.dot(xb, w1_ref[...], preferred_element_type=jnp.float32)
    asrc, adst = [], []
    for hh in range(HEADS):
        hb = h[:, hh * HID:(hh + 1) * HID]
        h1_ref[hh] = hb
        asrc.append(jnp.sum(hb * asw_ref[hh][None, :], axis=1))
        adst.append(jnp.sum(hb * adw_ref[hh][None, :], axis=1))
    a1_ref[...] = jnp.stack([jnp.stack(asrc), jnp.stack(adst)])


def _stage_a(x2d, W1, att_src1, att_dst1):
    blk = 2048
    grid = NUM_NODES // blk
    return pl.pallas_call(
        _stage_a_body,
        grid=(grid,),
        in_specs=[
            pl.BlockSpec((blk, T), lambda i: (i, 0)),
            pl.BlockSpec((T, HEADS * HID), lambda i: (0, 0)),
            pl.BlockSpec((HEADS, HID), lambda i: (0, 0)),
            pl.BlockSpec((HEADS, HID), lambda i: (0, 0)),
        ],
        out_specs=[
            pl.BlockSpec((HEADS, blk, HID), lambda i: (0, i, 0)),
            pl.BlockSpec((2, HEADS, blk), lambda i: (0, 0, i)),
        ],
        out_shape=[
            jax.ShapeDtypeStruct((HEADS, NUM_NODES, HID), jnp.float32),
            jax.ShapeDtypeStruct((2, HEADS, NUM_NODES), jnp.float32),
        ],
    )(x2d, W1, att_src1, att_dst1)


def _edge_chunk(tbl_ref, src_hbm, dst_hbm, ebase,
                asrc_t, adst_t, src_c, dst_c, rows, exrow,
                out_sh, den_sh, sem):
    pltpu.sync_copy(src_hbm.at[pl.ds(ebase, K)], src_c)
    pltpu.sync_copy(dst_hbm.at[pl.ds(ebase, K)], dst_c)
    pltpu.async_copy(tbl_ref.at[src_c], rows, sem).wait()
    col0 = jnp.full((L,), 0, jnp.int32)
    for i in range(K // L):
        s16 = src_c[pl.ds(i * L, L)]
        d16 = dst_c[pl.ds(i * L, L)]
        a = plsc.load_gather(asrc_t, (s16,)) + plsc.load_gather(adst_t, (d16,))
        a = jnp.where(a > 0, a, 0.2 * a)
        exv = jnp.exp(a)
        ridx = jnp.full((L,), i * L, jnp.int32) + lax.iota(jnp.int32, L)
        plsc.store_scatter(exrow, (ridx, col0), exv)
        for l in range(L):
            r = i * L + l
            exb = exv[jnp.full((L,), l, jnp.int32)]
            for c in range(HID // L):
                rows[r, pl.ds(c * L, L)] = rows[r, pl.ds(c * L, L)] * exb
    pltpu.sync_copy(rows, out_sh.at[dst_c], add=True)
    pltpu.sync_copy(exrow, den_sh.at[dst_c], add=True)


def _zero_exrow(exrow):
    z = jnp.zeros((L,), jnp.float32)
    for r in range(K):
        exrow[r, pl.ds(0, L)] = z


EPT1 = E // NS
NCHUNK1 = EPT1 // K

_SC_SCRATCH = lambda: [
    pltpu.VMEM_SHARED((NUM_NODES, HID), jnp.float32),
    pltpu.VMEM_SHARED((NUM_NODES, L), jnp.float32),
    pltpu.VMEM((NUM_NODES,), jnp.float32),
    pltpu.VMEM((NUM_NODES,), jnp.float32),
    pltpu.VMEM((K,), jnp.int32),
    pltpu.VMEM((K,), jnp.int32),
    pltpu.VMEM((K, HID), jnp.float32),
    pltpu.VMEM((K, L), jnp.float32),
    pltpu.SemaphoreType.DMA,
]


def _gat1_body(h1_hbm, asrc_hbm, adst_hbm, src_hbm, dst_hbm, zrow_hbm,
                zden_hbm, acc_hbm, den_hbm,
                out_sh, den_sh, asrc_t, adst_t, src_c, dst_c, rows,
                exrow, sem):
    cid = lax.axis_index("c")
    sid = lax.axis_index("s")
    _zero_exrow(exrow)
    stripe = pl.ds(sid * STRIPE, STRIPE)

    def head_pass(p, _):
        head = cid * 2 + p
        pltpu.sync_copy(zrow_hbm.at[stripe], out_sh.at[stripe])
        pltpu.sync_copy(zden_hbm.at[stripe], den_sh.at[stripe])
        pltpu.sync_copy(asrc_hbm.at[head], asrc_t)
        pltpu.sync_copy(adst_hbm.at[head], adst_t)
        plsc.subcore_barrier()

        def chunk(j, _c):
            _edge_chunk(h1_hbm.at[head], src_hbm, dst_hbm,
                        sid * EPT1 + j * K,
                        asrc_t, adst_t, src_c, dst_c, rows, exrow,
                        out_sh, den_sh, sem)
            return 0

        lax.fori_loop(0, NCHUNK1, chunk, 0)
        plsc.subcore_barrier()
        pltpu.sync_copy(out_sh.at[stripe], acc_hbm.at[head].at[stripe])
        pltpu.sync_copy(den_sh.at[stripe], den_hbm.at[head].at[stripe])
        plsc.subcore_barrier()
        return 0

    lax.fori_loop(0, 2, head_pass, 0)


def _stage_c_body(acc_ref, den_ref, b1_ref, w2_ref, as2_ref, ad2_ref,
                  h2_ref, a2_ref):
    parts = []
    for hh in range(HEADS):
        d = den_ref[hh, :, 0:1]
        parts.append(acc_ref[hh] / (d + 1e-16))
    hcat = jnp.concatenate(parts, axis=1) + b1_ref[...][None, :]
    hcat = _elu(hcat)
    h2 = jnp.dot(hcat, w2_ref[...], preferred_element_type=jnp.float32)
    h2_ref[...] = h2
    a2s = jnp.sum(h2 * as2_ref[0][None, :], axis=1)
    a2d = jnp.sum(h2 * ad2_ref[0][None, :], axis=1)
    a2_ref[...] = jnp.stack([a2s, a2d])


def _stage_c(acc1, den1, bias1, W2, att_src2, att_dst2):
    blk = 2048
    grid = NUM_NODES // blk
    return pl.pallas_call(
        _stage_c_body,
        grid=(grid,),
        in_specs=[
            pl.BlockSpec((HEADS, blk, HID), lambda i: (0, i, 0)),
            pl.BlockSpec((HEADS, blk, L), lambda i: (0, i, 0)),
            pl.BlockSpec((HEADS * HID,), lambda i: (0,)),
            pl.BlockSpec((HEADS * HID, HID), lambda i: (0, 0)),
            pl.BlockSpec((1, HID), lambda i: (0, 0)),
            pl.BlockSpec((1, HID), lambda i: (0, 0)),
        ],
        out_specs=[
            pl.BlockSpec((blk, HID), lambda i: (i, 0)),
            pl.BlockSpec((2, blk), lambda i: (0, i)),
        ],
        out_shape=[
            jax.ShapeDtypeStruct((NUM_NODES, HID), jnp.float32),
            jax.ShapeDtypeStruct((2, NUM_NODES), jnp.float32),
        ],
    )(acc1, den1, bias1, W2, att_src2, att_dst2)


EPT2 = E // (NC * NS)
NCHUNK2 = EPT2 // K


def _gat2_body(h2_hbm, asrc_hbm, adst_hbm, src_hbm, dst_hbm, zrow_hbm,
                zden_hbm, acc_hbm, den_hbm,
                out_sh, den_sh, asrc_t, adst_t, src_c, dst_c, rows,
                exrow, sem):
    cid = lax.axis_index("c")
    sid = lax.axis_index("s")
    wid = cid * NS + sid
    _zero_exrow(exrow)
    stripe = pl.ds(sid * STRIPE, STRIPE)
    pltpu.sync_copy(zrow_hbm.at[stripe], out_sh.at[stripe])
    pltpu.sync_copy(zden_hbm.at[stripe], den_sh.at[stripe])
    pltpu.sync_copy(asrc_hbm, asrc_t)
    pltpu.sync_copy(adst_hbm, adst_t)
    plsc.subcore_barrier()

    def chunk(j, _c):
        _edge_chunk(h2_hbm, src_hbm, dst_hbm, wid * EPT2 + j * K,
                    asrc_t, adst_t, src_c, dst_c, rows, exrow,
                    out_sh, den_sh, sem)
        return 0

    lax.fori_loop(0, NCHUNK2, chunk, 0)
    plsc.subcore_barrier()
    pltpu.sync_copy(out_sh.at[stripe], acc_hbm.at[cid].at[stripe])
    pltpu.sync_copy(den_sh.at[stripe], den_hbm.at[cid].at[stripe])


@functools.lru_cache(maxsize=None)
def _build_sc_kernels():
    mesh = plsc.VectorSubcoreMesh(core_axis_name="c", subcore_axis_name="s",
                                  num_cores=NC, num_subcores=NS)
    gat1 = pl.kernel(
        _gat1_body,
        out_type=(
            jax.ShapeDtypeStruct((HEADS, NUM_NODES, HID), jnp.float32),
            jax.ShapeDtypeStruct((HEADS, NUM_NODES, L), jnp.float32),
        ),
        mesh=mesh,
        scratch_types=_SC_SCRATCH(),
        compiler_params=pltpu.CompilerParams(
            needs_layout_passes=False, use_tc_tiling_on_sc=False),
    )
    gat2 = pl.kernel(
        _gat2_body,
        out_type=(
            jax.ShapeDtypeStruct((NC, NUM_NODES, HID), jnp.float32),
            jax.ShapeDtypeStruct((NC, NUM_NODES, L), jnp.float32),
        ),
        mesh=mesh,
        scratch_types=_SC_SCRATCH(),
        compiler_params=pltpu.CompilerParams(
            needs_layout_passes=False, use_tc_tiling_on_sc=False),
    )
    return gat1, gat2


def _stage_e_body(acc_ref, den_ref, b2_ref, lnw_ref, lnb_ref, ht_ref):
    acc = acc_ref[0] + acc_ref[1]
    den = den_ref[0, :, 0:1] + den_ref[1, :, 0:1]
    h = acc / (den + 1e-16) + b2_ref[...][None, :]
    h = _elu(h)
    mu = jnp.mean(h, axis=1, keepdims=True)
    var = jnp.mean((h - mu) ** 2, axis=1, keepdims=True)
    h = (h - mu) * lax.rsqrt(var + 1e-5) * lnw_ref[...][None, :] \
        + lnb_ref[...][None, :]
    ht_ref[...] = h.T


def _stage_e(acc2, den2, bias2, ln_w, ln_b):
    blk = 1024
    grid = NUM_NODES // blk
    return pl.pallas_call(
        _stage_e_body,
        grid=(grid,),
        in_specs=[
            pl.BlockSpec((NC, blk, HID), lambda i: (0, i, 0)),
            pl.BlockSpec((NC, blk, L), lambda i: (0, i, 0)),
            pl.BlockSpec((HID,), lambda i: (0,)),
            pl.BlockSpec((HID,), lambda i: (0,)),
            pl.BlockSpec((HID,), lambda i: (0,)),
        ],
        out_specs=pl.BlockSpec((HID, blk), lambda i: (0, i)),
        out_shape=jax.ShapeDtypeStruct((HID, NUM_NODES), jnp.float32),
    )(acc2, den2, bias2, ln_w, ln_b)


def _stage_f_body(ht_ref, wi1_ref, wh1_ref, bi1_ref, bh1_ref,
                  wi2_ref, wh2_ref, bi2_ref, bh2_ref, fcw_ref, fcb_ref,
                  out_ref):
    def gru_step(x, hprev, wi_ref, wh_ref, bi_ref, bh_ref):
        gi = jnp.dot(x, wi_ref[...], preferred_element_type=jnp.float32) \
            + bi_ref[...][None, :]
        gh = jnp.dot(hprev, wh_ref[...], preferred_element_type=jnp.float32) \
            + bh_ref[...][None, :]
        r = jax.nn.sigmoid(gi[:, :HID] + gh[:, :HID])
        z = jax.nn.sigmoid(gi[:, HID:2 * HID] + gh[:, HID:2 * HID])
        n = jnp.tanh(gi[:, 2 * HID:] + r * gh[:, 2 * HID:])
        return (1.0 - z) * n + z * hprev

    def step(t, carry):
        h1, h2 = carry
        xt = ht_ref[t]
        h1 = gru_step(xt, h1, wi1_ref, wh1_ref, bi1_ref, bh1_ref)
        h2 = gru_step(h1, h2, wi2_ref, wh2_ref, bi2_ref, bh2_ref)
        return (h1, h2)

    h0 = (jnp.zeros((B, HID), jnp.float32), jnp.zeros((B, HID), jnp.float32))
    h1, h2 = lax.fori_loop(0, HID, step, h0)
    out_ref[...] = jnp.dot(h2, fcw_ref[...],
                           preferred_element_type=jnp.float32) \
        + fcb_ref[...][None, :]


def _stage_f(ht3, Wi1T, Wh1T, bi1, bh1, Wi2T, Wh2T, bi2, bh2, fc_wT, fc_b):
    return pl.pallas_call(
        _stage_f_body,
        out_shape=jax.ShapeDtypeStruct((B, OUT), jnp.float32),
    )(ht3, Wi1T, Wh1T, bi1, bh1, Wi2T, Wh2T, bi2, bh2, fc_wT, fc_b)


def kernel(x, edge_index, W1, att_src1, att_dst1, bias1, W2, att_src2,
           att_dst2, bias2, ln_w, ln_b, Wi1, Wh1, bi1, bh1, Wi2, Wh2, bi2,
           bh2, fc_w, fc_b):
    x2d = x.reshape(NUM_NODES, T)
    src = edge_index[0]
    dst = edge_index[1]
    zrow = jnp.zeros((NUM_NODES, HID), jnp.float32)
    zden = jnp.zeros((NUM_NODES, L), jnp.float32)

    h1, a1 = _stage_a(x2d, W1, att_src1, att_dst1)

    _gat1_edges, _gat2_edges = _build_sc_kernels()

    acc1, den1 = _gat1_edges(h1, a1[0], a1[1], src, dst, zrow, zden)

    h2, a2 = _stage_c(acc1, den1, bias1, W2, att_src2, att_dst2)

    acc2, den2 = _gat2_edges(h2, a2[0], a2[1], src, dst, zrow, zden)

    ht = _stage_e(acc2, den2, bias2, ln_w, ln_b)
    ht3 = ht.reshape(HID, B, N)

    return _stage_f(ht3, Wi1.T, Wh1.T, bi1, bh1, Wi2.T, Wh2.T, bi2, bh2,
                    fc_w.T, fc_b)


# speedup vs baseline: 48.4849x; 1.2664x over previous
import functools

import jax
import jax.numpy as jnp
from jax import lax
from jax.experimental import pallas as pl
from jax.experimental.pallas import tpu as pltpu
from jax.experimental.pallas import tpu_sc as plsc

B, N, T = 512, 32, 12
HID, HEADS, OUT = 64, 4, 12
NUM_NODES = B * N
E = 262144

NC, NS, L = 2, 16, 16
K = 64
STRIPE = NUM_NODES // NS


def _elu(x):
    return jnp.where(x > 0, x, jnp.exp(jnp.minimum(x, 0.0)) - 1.0)


def _stage_a_body(x_ref, w1_ref, asw_ref, adw_ref, h1_ref, a1_ref):
    xb = x_ref[...]
    h = jnp.dot(xb, w1_ref[...], preferred_element_type=jnp.float32)
    asrc, adst = [], []
    for hh in range(HEADS):
        hb = h[:, hh * HID:(hh + 1) * HID]
        h1_ref[hh] = hb
        asrc.append(jnp.sum(hb * asw_ref[hh][None, :], axis=1))
        adst.append(jnp.sum(hb * adw_ref[hh][None, :], axis=1))
    a1_ref[...] = jnp.stack([jnp.stack(asrc), jnp.stack(adst)])


def _stage_a(x2d, W1, att_src1, att_dst1):
    blk = 2048
    grid = NUM_NODES // blk
    return pl.pallas_call(
        _stage_a_body,
        grid=(grid,),
        in_specs=[
            pl.BlockSpec((blk, T), lambda i: (i, 0)),
            pl.BlockSpec((T, HEADS * HID), lambda i: (0, 0)),
            pl.BlockSpec((HEADS, HID), lambda i: (0, 0)),
            pl.BlockSpec((HEADS, HID), lambda i: (0, 0)),
        ],
        out_specs=[
            pl.BlockSpec((HEADS, blk, HID), lambda i: (0, i, 0)),
            pl.BlockSpec((2, HEADS, blk), lambda i: (0, 0, i)),
        ],
        out_shape=[
            jax.ShapeDtypeStruct((HEADS, NUM_NODES, HID), jnp.float32),
            jax.ShapeDtypeStruct((2, HEADS, NUM_NODES), jnp.float32),
        ],
    )(x2d, W1, att_src1, att_dst1)


def _issue_idx(src_hbm, dst_hbm, ebase, src_cb, dst_cb, semi):
    pltpu.async_copy(src_hbm.at[pl.ds(ebase, K)], src_cb, semi)
    pltpu.async_copy(dst_hbm.at[pl.ds(ebase, K)], dst_cb, semi)


def _wait_idx(src_hbm, dst_hbm, src_cb, dst_cb, semi):
    pltpu.make_async_copy(src_hbm.at[pl.ds(0, K)], src_cb, semi).wait()
    pltpu.make_async_copy(dst_hbm.at[pl.ds(0, K)], dst_cb, semi).wait()


def _scale_and_scatter(asrc_t, adst_t, src_cb, dst_cb, rowsb, exrow,
                       out_sh, den_sh):
    col0 = jnp.full((L,), 0, jnp.int32)
    for i in range(K // L):
        s16 = src_cb[pl.ds(i * L, L)]
        d16 = dst_cb[pl.ds(i * L, L)]
        a = plsc.load_gather(asrc_t, (s16,)) + plsc.load_gather(adst_t, (d16,))
        a = jnp.where(a > 0, a, 0.2 * a)
        exv = jnp.exp(a)
        ridx = jnp.full((L,), i * L, jnp.int32) + lax.iota(jnp.int32, L)
        plsc.store_scatter(exrow, (ridx, col0), exv)
        for l in range(L):
            r = i * L + l
            exb = exv[jnp.full((L,), l, jnp.int32)]
            for c in range(HID // L):
                rowsb[r, pl.ds(c * L, L)] = rowsb[r, pl.ds(c * L, L)] * exb
    pltpu.sync_copy(rowsb, out_sh.at[dst_cb], add=True)
    pltpu.sync_copy(exrow, den_sh.at[dst_cb], add=True)


def _edge_pass(tbl_ref, src_hbm, dst_hbm, tile_base, nch,
               asrc_t, adst_t, src_c2, dst_c2, rows2, exrow,
               out_sh, den_sh, semi, semg):
    sc = (src_c2.at[0], src_c2.at[1])
    dc = (dst_c2.at[0], dst_c2.at[1])
    rw = (rows2.at[0], rows2.at[1])
    _issue_idx(src_hbm, dst_hbm, tile_base, sc[0], dc[0], semi[0])
    _issue_idx(src_hbm, dst_hbm, tile_base + K, sc[1], dc[1], semi[1])
    _wait_idx(src_hbm, dst_hbm, sc[0], dc[0], semi[0])
    pltpu.async_copy(tbl_ref.at[sc[0]], rw[0], semg[0])

    def it(jj, _):
        for b in (0, 1):
            ob = 1 - b
            pltpu.make_async_copy(tbl_ref.at[sc[b]], rw[b], semg[b]).wait()
            _wait_idx(src_hbm, dst_hbm, sc[ob], dc[ob], semi[ob])
            pltpu.async_copy(tbl_ref.at[sc[ob]], rw[ob], semg[ob])
            _scale_and_scatter(asrc_t, adst_t, sc[b], dc[b], rw[b], exrow,
                               out_sh, den_sh)
            nxt = lax.rem(2 * jj + b + 2, nch)
            _issue_idx(src_hbm, dst_hbm, tile_base + nxt * K,
                       sc[b], dc[b], semi[b])
        return 0

    lax.fori_loop(0, nch // 2, it, 0)
    pltpu.make_async_copy(tbl_ref.at[sc[0]], rw[0], semg[0]).wait()
    _wait_idx(src_hbm, dst_hbm, sc[1], dc[1], semi[1])


def _zero_exrow(exrow):
    z = jnp.zeros((L,), jnp.float32)
    for r in range(K):
        exrow[r, pl.ds(0, L)] = z


EPT1 = E // NS
NCHUNK1 = EPT1 // K

_SC_SCRATCH = lambda: [
    pltpu.VMEM_SHARED((NUM_NODES, HID), jnp.float32),
    pltpu.VMEM_SHARED((NUM_NODES, L), jnp.float32),
    pltpu.VMEM((NUM_NODES,), jnp.float32),
    pltpu.VMEM((NUM_NODES,), jnp.float32),
    pltpu.VMEM((2, K), jnp.int32),
    pltpu.VMEM((2, K), jnp.int32),
    pltpu.VMEM((2, K, HID), jnp.float32),
    pltpu.VMEM((K, L), jnp.float32),
    pltpu.SemaphoreType.DMA,
    pltpu.SemaphoreType.DMA,
    pltpu.SemaphoreType.DMA,
    pltpu.SemaphoreType.DMA,
]


def _gat1_body(h1_hbm, asrc_hbm, adst_hbm, src_hbm, dst_hbm, zrow_hbm,
                zden_hbm, acc_hbm, den_hbm,
                out_sh, den_sh, asrc_t, adst_t, src_c2, dst_c2, rows2,
                exrow, semi0, semi1, semg0, semg1):
    cid = lax.axis_index("c")
    sid = lax.axis_index("s")
    _zero_exrow(exrow)
    stripe = pl.ds(sid * STRIPE, STRIPE)

    def head_pass(p, _):
        head = cid * 2 + p
        pltpu.sync_copy(zrow_hbm.at[stripe], out_sh.at[stripe])
        pltpu.sync_copy(zden_hbm.at[stripe], den_sh.at[stripe])
        pltpu.sync_copy(asrc_hbm.at[head], asrc_t)
        pltpu.sync_copy(adst_hbm.at[head], adst_t)
        plsc.subcore_barrier()
        _edge_pass(h1_hbm.at[head], src_hbm, dst_hbm, sid * EPT1, NCHUNK1,
                   asrc_t, adst_t, src_c2, dst_c2, rows2, exrow,
                   out_sh, den_sh, (semi0, semi1), (semg0, semg1))
        plsc.subcore_barrier()
        pltpu.sync_copy(out_sh.at[stripe], acc_hbm.at[head].at[stripe])
        pltpu.sync_copy(den_sh.at[stripe], den_hbm.at[head].at[stripe])
        plsc.subcore_barrier()
        return 0

    lax.fori_loop(0, 2, head_pass, 0)


def _stage_c_body(acc_ref, den_ref, b1_ref, w2_ref, as2_ref, ad2_ref,
                  h2_ref, a2_ref):
    parts = []
    for hh in range(HEADS):
        d = den_ref[hh, :, 0:1]
        parts.append(acc_ref[hh] / (d + 1e-16))
    hcat = jnp.concatenate(parts, axis=1) + b1_ref[...][None, :]
    hcat = _elu(hcat)
    h2 = jnp.dot(hcat, w2_ref[...], preferred_element_type=jnp.float32)
    h2_ref[...] = h2
    a2s = jnp.sum(h2 * as2_ref[0][None, :], axis=1)
    a2d = jnp.sum(h2 * ad2_ref[0][None, :], axis=1)
    a2_ref[...] = jnp.stack([a2s, a2d])


def _stage_c(acc1, den1, bias1, W2, att_src2, att_dst2):
    blk = 2048
    grid = NUM_NODES // blk
    return pl.pallas_call(
        _stage_c_body,
        grid=(grid,),
        in_specs=[
            pl.BlockSpec((HEADS, blk, HID), lambda i: (0, i, 0)),
            pl.BlockSpec((HEADS, blk, L), lambda i: (0, i, 0)),
            pl.BlockSpec((HEADS * HID,), lambda i: (0,)),
            pl.BlockSpec((HEADS * HID, HID), lambda i: (0, 0)),
            pl.BlockSpec((1, HID), lambda i: (0, 0)),
            pl.BlockSpec((1, HID), lambda i: (0, 0)),
        ],
        out_specs=[
            pl.BlockSpec((blk, HID), lambda i: (i, 0)),
            pl.BlockSpec((2, blk), lambda i: (0, i)),
        ],
        out_shape=[
            jax.ShapeDtypeStruct((NUM_NODES, HID), jnp.float32),
            jax.ShapeDtypeStruct((2, NUM_NODES), jnp.float32),
        ],
    )(acc1, den1, bias1, W2, att_src2, att_dst2)


EPT2 = E // (NC * NS)
NCHUNK2 = EPT2 // K


def _gat2_body(h2_hbm, asrc_hbm, adst_hbm, src_hbm, dst_hbm, zrow_hbm,
                zden_hbm, acc_hbm, den_hbm,
                out_sh, den_sh, asrc_t, adst_t, src_c2, dst_c2, rows2,
                exrow, semi0, semi1, semg0, semg1):
    cid = lax.axis_index("c")
    sid = lax.axis_index("s")
    wid = cid * NS + sid
    _zero_exrow(exrow)
    stripe = pl.ds(sid * STRIPE, STRIPE)
    pltpu.sync_copy(zrow_hbm.at[stripe], out_sh.at[stripe])
    pltpu.sync_copy(zden_hbm.at[stripe], den_sh.at[stripe])
    pltpu.sync_copy(asrc_hbm, asrc_t)
    pltpu.sync_copy(adst_hbm, adst_t)
    plsc.subcore_barrier()
    _edge_pass(h2_hbm, src_hbm, dst_hbm, wid * EPT2, NCHUNK2,
               asrc_t, adst_t, src_c2, dst_c2, rows2, exrow,
               out_sh, den_sh, (semi0, semi1), (semg0, semg1))
    plsc.subcore_barrier()
    pltpu.sync_copy(out_sh.at[stripe], acc_hbm.at[cid].at[stripe])
    pltpu.sync_copy(den_sh.at[stripe], den_hbm.at[cid].at[stripe])


@functools.lru_cache(maxsize=None)
def _build_sc_kernels():
    mesh = plsc.VectorSubcoreMesh(core_axis_name="c", subcore_axis_name="s",
                                  num_cores=NC, num_subcores=NS)
    gat1 = pl.kernel(
        _gat1_body,
        out_type=(
            jax.ShapeDtypeStruct((HEADS, NUM_NODES, HID), jnp.float32),
            jax.ShapeDtypeStruct((HEADS, NUM_NODES, L), jnp.float32),
        ),
        mesh=mesh,
        scratch_types=_SC_SCRATCH(),
        compiler_params=pltpu.CompilerParams(
            needs_layout_passes=False, use_tc_tiling_on_sc=False),
    )
    gat2 = pl.kernel(
        _gat2_body,
        out_type=(
            jax.ShapeDtypeStruct((NC, NUM_NODES, HID), jnp.float32),
            jax.ShapeDtypeStruct((NC, NUM_NODES, L), jnp.float32),
        ),
        mesh=mesh,
        scratch_types=_SC_SCRATCH(),
        compiler_params=pltpu.CompilerParams(
            needs_layout_passes=False, use_tc_tiling_on_sc=False),
    )
    return gat1, gat2


def _stage_e_body(acc_ref, den_ref, b2_ref, lnw_ref, lnb_ref, ht_ref):
    acc = acc_ref[0] + acc_ref[1]
    den = den_ref[0, :, 0:1] + den_ref[1, :, 0:1]
    h = acc / (den + 1e-16) + b2_ref[...][None, :]
    h = _elu(h)
    mu = jnp.mean(h, axis=1, keepdims=True)
    var = jnp.mean((h - mu) ** 2, axis=1, keepdims=True)
    h = (h - mu) * lax.rsqrt(var + 1e-5) * lnw_ref[...][None, :] \
        + lnb_ref[...][None, :]
    ht_ref[...] = h.T


def _stage_e(acc2, den2, bias2, ln_w, ln_b):
    blk = 1024
    grid = NUM_NODES // blk
    return pl.pallas_call(
        _stage_e_body,
        grid=(grid,),
        in_specs=[
            pl.BlockSpec((NC, blk, HID), lambda i: (0, i, 0)),
            pl.BlockSpec((NC, blk, L), lambda i: (0, i, 0)),
            pl.BlockSpec((HID,), lambda i: (0,)),
            pl.BlockSpec((HID,), lambda i: (0,)),
            pl.BlockSpec((HID,), lambda i: (0,)),
        ],
        out_specs=pl.BlockSpec((HID, blk), lambda i: (0, i)),
        out_shape=jax.ShapeDtypeStruct((HID, NUM_NODES), jnp.float32),
    )(acc2, den2, bias2, ln_w, ln_b)


def _stage_f_body(ht_ref, wi1_ref, wh1_ref, bi1_ref, bh1_ref,
                  wi2_ref, wh2_ref, bi2_ref, bh2_ref, fcw_ref, fcb_ref,
                  out_ref):
    def gru_step(x, hprev, wi_ref, wh_ref, bi_ref, bh_ref):
        gi = jnp.dot(x, wi_ref[...], preferred_element_type=jnp.float32) \
            + bi_ref[...][None, :]
        gh = jnp.dot(hprev, wh_ref[...], preferred_element_type=jnp.float32) \
            + bh_ref[...][None, :]
        r = jax.nn.sigmoid(gi[:, :HID] + gh[:, :HID])
        z = jax.nn.sigmoid(gi[:, HID:2 * HID] + gh[:, HID:2 * HID])
        n = jnp.tanh(gi[:, 2 * HID:] + r * gh[:, 2 * HID:])
        return (1.0 - z) * n + z * hprev

    def step(t, carry):
        h1, h2 = carry
        xt = ht_ref[t]
        h1 = gru_step(xt, h1, wi1_ref, wh1_ref, bi1_ref, bh1_ref)
        h2 = gru_step(h1, h2, wi2_ref, wh2_ref, bi2_ref, bh2_ref)
        return (h1, h2)

    h0 = (jnp.zeros((B, HID), jnp.float32), jnp.zeros((B, HID), jnp.float32))
    h1, h2 = lax.fori_loop(0, HID, step, h0)
    out_ref[...] = jnp.dot(h2, fcw_ref[...],
                           preferred_element_type=jnp.float32) \
        + fcb_ref[...][None, :]


def _stage_f(ht3, Wi1T, Wh1T, bi1, bh1, Wi2T, Wh2T, bi2, bh2, fc_wT, fc_b):
    return pl.pallas_call(
        _stage_f_body,
        out_shape=jax.ShapeDtypeStruct((B, OUT), jnp.float32),
    )(ht3, Wi1T, Wh1T, bi1, bh1, Wi2T, Wh2T, bi2, bh2, fc_wT, fc_b)


def kernel(x, edge_index, W1, att_src1, att_dst1, bias1, W2, att_src2,
           att_dst2, bias2, ln_w, ln_b, Wi1, Wh1, bi1, bh1, Wi2, Wh2, bi2,
           bh2, fc_w, fc_b):
    x2d = x.reshape(NUM_NODES, T)
    src = edge_index[0]
    dst = edge_index[1]
    zrow = jnp.zeros((NUM_NODES, HID), jnp.float32)
    zden = jnp.zeros((NUM_NODES, L), jnp.float32)

    h1, a1 = _stage_a(x2d, W1, att_src1, att_dst1)

    _gat1_edges, _gat2_edges = _build_sc_kernels()

    acc1, den1 = _gat1_edges(h1, a1[0], a1[1], src, dst, zrow, zden)

    h2, a2 = _stage_c(acc1, den1, bias1, W2, att_src2, att_dst2)

    acc2, den2 = _gat2_edges(h2, a2[0], a2[1], src, dst, zrow, zden)

    ht = _stage_e(acc2, den2, bias2, ln_w, ln_b)
    ht3 = ht.reshape(HID, B, N)

    return _stage_f(ht3, Wi1.T, Wh1.T, bi1, bh1, Wi2.T, Wh2.T, bi2, bh2,
                    fc_w.T, fc_b)


# speedup vs baseline: 55.6934x; 1.1487x over previous
import functools

import jax
import jax.numpy as jnp
from jax import lax
from jax.experimental import pallas as pl
from jax.experimental.pallas import tpu as pltpu
from jax.experimental.pallas import tpu_sc as plsc

B, N, T = 512, 32, 12
HID, HEADS, OUT = 64, 4, 12
NUM_NODES = B * N
E = 262144

NC, NS, L = 2, 16, 16
K = 64
STRIPE = NUM_NODES // NS


def _elu(x):
    return jnp.where(x > 0, x, jnp.exp(jnp.minimum(x, 0.0)) - 1.0)


def _stage_a_body(x_ref, w1_ref, asw_ref, adw_ref, h1_ref, a1_ref):
    xb = x_ref[...]
    h = jnp.dot(xb, w1_ref[...], preferred_element_type=jnp.float32)
    asrc, adst = [], []
    for hh in range(HEADS):
        hb = h[:, hh * HID:(hh + 1) * HID]
        h1_ref[hh] = hb
        asrc.append(jnp.sum(hb * asw_ref[hh][None, :], axis=1))
        adst.append(jnp.sum(hb * adw_ref[hh][None, :], axis=1))
    a1_ref[...] = jnp.stack([jnp.stack(asrc), jnp.stack(adst)])


def _stage_a(x2d, W1, att_src1, att_dst1):
    blk = 2048
    grid = NUM_NODES // blk
    return pl.pallas_call(
        _stage_a_body,
        grid=(grid,),
        in_specs=[
            pl.BlockSpec((blk, T), lambda i: (i, 0)),
            pl.BlockSpec((T, HEADS * HID), lambda i: (0, 0)),
            pl.BlockSpec((HEADS, HID), lambda i: (0, 0)),
            pl.BlockSpec((HEADS, HID), lambda i: (0, 0)),
        ],
        out_specs=[
            pl.BlockSpec((HEADS, blk, HID), lambda i: (0, i, 0)),
            pl.BlockSpec((2, HEADS, blk), lambda i: (0, 0, i)),
        ],
        out_shape=[
            jax.ShapeDtypeStruct((HEADS, NUM_NODES, HID), jnp.float32),
            jax.ShapeDtypeStruct((2, HEADS, NUM_NODES), jnp.float32),
        ],
    )(x2d, W1, att_src1, att_dst1)


def _issue_idx(src_hbm, dst_hbm, ebase, src_cb, dst_cb, semi):
    pltpu.async_copy(src_hbm.at[pl.ds(ebase, K)], src_cb, semi)
    pltpu.async_copy(dst_hbm.at[pl.ds(ebase, K)], dst_cb, semi)


def _wait_idx(src_hbm, dst_hbm, src_cb, dst_cb, semi):
    pltpu.make_async_copy(src_hbm.at[pl.ds(0, K)], src_cb, semi).wait()
    pltpu.make_async_copy(dst_hbm.at[pl.ds(0, K)], dst_cb, semi).wait()


def _scale(asrc_t, adst_t, src_cb, dst_cb, rowsb, exrowb):
    col0 = jnp.full((L,), 0, jnp.int32)
    for i in range(K // L):
        s16 = src_cb[pl.ds(i * L, L)]
        d16 = dst_cb[pl.ds(i * L, L)]
        a = plsc.load_gather(asrc_t, (s16,)) + plsc.load_gather(adst_t, (d16,))
        a = jnp.where(a > 0, a, 0.2 * a)
        exv = jnp.exp(a)
        ridx = jnp.full((L,), i * L, jnp.int32) + lax.iota(jnp.int32, L)
        plsc.store_scatter(exrowb, (ridx, col0), exv)
        for l in range(L):
            r = i * L + l
            exb = exv[jnp.full((L,), l, jnp.int32)]
            for c in range(HID // L):
                rowsb[r, pl.ds(c * L, L)] = rowsb[r, pl.ds(c * L, L)] * exb


def _edge_pass(tbl_ref, src_hbm, dst_hbm, tile_base, nch,
               asrc_t, adst_t, src_c2, dst_c2, dsc2, rows2, exrow2,
               out_sh, den_sh, semi, semg, sems, semd):
    sc = (src_c2.at[0], src_c2.at[1])
    dc = (dst_c2.at[0], dst_c2.at[1])
    ds = (dsc2.at[0], dsc2.at[1])
    rw = (rows2.at[0], rows2.at[1])
    ex = (exrow2.at[0], exrow2.at[1])

    def seg(q, c1base, c2base, first):
        o = 1 - q
        pltpu.make_async_copy(tbl_ref.at[sc[q]], rw[q], semg[q]).wait()
        _wait_idx(src_hbm, dst_hbm, sc[o], dc[o], semi[o])
        if not first:
            pltpu.make_async_copy(rw[o], out_sh.at[ds[o]], sems[o]).wait()
            pltpu.make_async_copy(ex[o], den_sh.at[ds[o]], sems[o]).wait()
            pltpu.async_copy(dst_hbm.at[pl.ds(c1base, K)], ds[o], semd[o])
        pltpu.async_copy(tbl_ref.at[sc[o]], rw[o], semg[o])
        _scale(asrc_t, adst_t, sc[q], dc[q], rw[q], ex[q])
        pltpu.make_async_copy(dst_hbm.at[pl.ds(0, K)], ds[q], semd[q]).wait()
        pltpu.async_copy(rw[q], out_sh.at[ds[q]], sems[q], add=True)
        pltpu.async_copy(ex[q], den_sh.at[ds[q]], sems[q], add=True)
        _issue_idx(src_hbm, dst_hbm, c2base, sc[q], dc[q], semi[q])

    _issue_idx(src_hbm, dst_hbm, tile_base, sc[0], dc[0], semi[0])
    _issue_idx(src_hbm, dst_hbm, tile_base + K, sc[1], dc[1], semi[1])
    pltpu.async_copy(dst_hbm.at[pl.ds(tile_base, K)], ds[0], semd[0])
    pltpu.async_copy(dst_hbm.at[pl.ds(tile_base + K, K)], ds[1], semd[1])
    _wait_idx(src_hbm, dst_hbm, sc[0], dc[0], semi[0])
    pltpu.async_copy(tbl_ref.at[sc[0]], rw[0], semg[0])
    seg(0, tile_base + K, tile_base + 2 * K, True)
    seg(1, tile_base + 2 * K, tile_base + 3 * K, False)

    def it(jj, _):
        c = 2 * jj + 2
        seg(0, tile_base + lax.rem(c + 1, nch) * K,
            tile_base + lax.rem(c + 2, nch) * K, False)
        seg(1, tile_base + lax.rem(c + 2, nch) * K,
            tile_base + lax.rem(c + 3, nch) * K, False)
        return 0

    lax.fori_loop(0, (nch - 2) // 2, it, 0)
    pltpu.make_async_copy(tbl_ref.at[sc[0]], rw[0], semg[0]).wait()
    _wait_idx(src_hbm, dst_hbm, sc[1], dc[1], semi[1])
    pltpu.make_async_copy(rw[1], out_sh.at[ds[1]], sems[1]).wait()
    pltpu.make_async_copy(ex[1], den_sh.at[ds[1]], sems[1]).wait()
    pltpu.make_async_copy(dst_hbm.at[pl.ds(0, K)], ds[0], semd[0]).wait()


def _zero_exrow(exrow2):
    z = jnp.zeros((L,), jnp.float32)
    for b in range(2):
        for r in range(K):
            exrow2[b, r, pl.ds(0, L)] = z


EPT1 = E // NS
NCHUNK1 = EPT1 // K

_SC_SCRATCH = lambda: [
    pltpu.VMEM_SHARED((NUM_NODES, HID), jnp.float32),
    pltpu.VMEM_SHARED((NUM_NODES, L), jnp.float32),
    pltpu.VMEM((NUM_NODES,), jnp.float32),
    pltpu.VMEM((NUM_NODES,), jnp.float32),
    pltpu.VMEM((2, K), jnp.int32),
    pltpu.VMEM((2, K), jnp.int32),
    pltpu.VMEM((2, K), jnp.int32),
    pltpu.VMEM((2, K, HID), jnp.float32),
    pltpu.VMEM((2, K, L), jnp.float32),
    pltpu.SemaphoreType.DMA,
    pltpu.SemaphoreType.DMA,
    pltpu.SemaphoreType.DMA,
    pltpu.SemaphoreType.DMA,
    pltpu.SemaphoreType.DMA,
    pltpu.SemaphoreType.DMA,
    pltpu.SemaphoreType.DMA,
    pltpu.SemaphoreType.DMA,
]


def _gat1_body(h1_hbm, asrc_hbm, adst_hbm, src_hbm, dst_hbm, zrow_hbm,
                zden_hbm, acc_hbm, den_hbm,
                out_sh, den_sh, asrc_t, adst_t, src_c2, dst_c2, dsc2, rows2,
                exrow2, semi0, semi1, semg0, semg1, sems0, sems1,
                semd0, semd1):
    cid = lax.axis_index("c")
    sid = lax.axis_index("s")
    _zero_exrow(exrow2)
    stripe = pl.ds(sid * STRIPE, STRIPE)

    def head_pass(p, _):
        head = cid * 2 + p
        pltpu.sync_copy(zrow_hbm.at[stripe], out_sh.at[stripe])
        pltpu.sync_copy(zden_hbm.at[stripe], den_sh.at[stripe])
        pltpu.sync_copy(asrc_hbm.at[head], asrc_t)
        pltpu.sync_copy(adst_hbm.at[head], adst_t)
        plsc.subcore_barrier()
        _edge_pass(h1_hbm.at[head], src_hbm, dst_hbm, sid * EPT1, NCHUNK1,
                   asrc_t, adst_t, src_c2, dst_c2, dsc2, rows2, exrow2,
                   out_sh, den_sh, (semi0, semi1), (semg0, semg1),
                   (sems0, sems1), (semd0, semd1))
        plsc.subcore_barrier()
        pltpu.sync_copy(out_sh.at[stripe], acc_hbm.at[head].at[stripe])
        pltpu.sync_copy(den_sh.at[stripe], den_hbm.at[head].at[stripe])
        plsc.subcore_barrier()
        return 0

    lax.fori_loop(0, 2, head_pass, 0)


def _stage_c_body(acc_ref, den_ref, b1_ref, w2_ref, as2_ref, ad2_ref,
                  h2_ref, a2_ref):
    parts = []
    for hh in range(HEADS):
        d = den_ref[hh, :, 0:1]
        parts.append(acc_ref[hh] / (d + 1e-16))
    hcat = jnp.concatenate(parts, axis=1) + b1_ref[...][None, :]
    hcat = _elu(hcat)
    h2 = jnp.dot(hcat, w2_ref[...], preferred_element_type=jnp.float32)
    h2_ref[...] = h2
    a2s = jnp.sum(h2 * as2_ref[0][None, :], axis=1)
    a2d = jnp.sum(h2 * ad2_ref[0][None, :], axis=1)
    a2_ref[...] = jnp.stack([a2s, a2d])


def _stage_c(acc1, den1, bias1, W2, att_src2, att_dst2):
    blk = 2048
    grid = NUM_NODES // blk
    return pl.pallas_call(
        _stage_c_body,
        grid=(grid,),
        in_specs=[
            pl.BlockSpec((HEADS, blk, HID), lambda i: (0, i, 0)),
            pl.BlockSpec((HEADS, blk, L), lambda i: (0, i, 0)),
            pl.BlockSpec((HEADS * HID,), lambda i: (0,)),
            pl.BlockSpec((HEADS * HID, HID), lambda i: (0, 0)),
            pl.BlockSpec((1, HID), lambda i: (0, 0)),
            pl.BlockSpec((1, HID), lambda i: (0, 0)),
        ],
        out_specs=[
            pl.BlockSpec((blk, HID), lambda i: (i, 0)),
            pl.BlockSpec((2, blk), lambda i: (0, i)),
        ],
        out_shape=[
            jax.ShapeDtypeStruct((NUM_NODES, HID), jnp.float32),
            jax.ShapeDtypeStruct((2, NUM_NODES), jnp.float32),
        ],
    )(acc1, den1, bias1, W2, att_src2, att_dst2)


EPT2 = E // (NC * NS)
NCHUNK2 = EPT2 // K


def _gat2_body(h2_hbm, asrc_hbm, adst_hbm, src_hbm, dst_hbm, zrow_hbm,
                zden_hbm, acc_hbm, den_hbm,
                out_sh, den_sh, asrc_t, adst_t, src_c2, dst_c2, dsc2, rows2,
                exrow2, semi0, semi1, semg0, semg1, sems0, sems1,
                semd0, semd1):
    cid = lax.axis_index("c")
    sid = lax.axis_index("s")
    wid = cid * NS + sid
    _zero_exrow(exrow2)
    stripe = pl.ds(sid * STRIPE, STRIPE)
    pltpu.sync_copy(zrow_hbm.at[stripe], out_sh.at[stripe])
    pltpu.sync_copy(zden_hbm.at[stripe], den_sh.at[stripe])
    pltpu.sync_copy(asrc_hbm, asrc_t)
    pltpu.sync_copy(adst_hbm, adst_t)
    plsc.subcore_barrier()
    _edge_pass(h2_hbm, src_hbm, dst_hbm, wid * EPT2, NCHUNK2,
               asrc_t, adst_t, src_c2, dst_c2, dsc2, rows2, exrow2,
               out_sh, den_sh, (semi0, semi1), (semg0, semg1),
               (sems0, sems1), (semd0, semd1))
    plsc.subcore_barrier()
    pltpu.sync_copy(out_sh.at[stripe], acc_hbm.at[cid].at[stripe])
    pltpu.sync_copy(den_sh.at[stripe], den_hbm.at[cid].at[stripe])


@functools.lru_cache(maxsize=None)
def _build_sc_kernels():
    mesh = plsc.VectorSubcoreMesh(core_axis_name="c", subcore_axis_name="s",
                                  num_cores=NC, num_subcores=NS)
    gat1 = pl.kernel(
        _gat1_body,
        out_type=(
            jax.ShapeDtypeStruct((HEADS, NUM_NODES, HID), jnp.float32),
            jax.ShapeDtypeStruct((HEADS, NUM_NODES, L), jnp.float32),
        ),
        mesh=mesh,
        scratch_types=_SC_SCRATCH(),
        compiler_params=pltpu.CompilerParams(
            needs_layout_passes=False, use_tc_tiling_on_sc=False),
    )
    gat2 = pl.kernel(
        _gat2_body,
        out_type=(
            jax.ShapeDtypeStruct((NC, NUM_NODES, HID), jnp.float32),
            jax.ShapeDtypeStruct((NC, NUM_NODES, L), jnp.float32),
        ),
        mesh=mesh,
        scratch_types=_SC_SCRATCH(),
        compiler_params=pltpu.CompilerParams(
            needs_layout_passes=False, use_tc_tiling_on_sc=False),
    )
    return gat1, gat2


def _stage_e_body(acc_ref, den_ref, b2_ref, lnw_ref, lnb_ref, ht_ref):
    acc = acc_ref[0] + acc_ref[1]
    den = den_ref[0, :, 0:1] + den_ref[1, :, 0:1]
    h = acc / (den + 1e-16) + b2_ref[...][None, :]
    h = _elu(h)
    mu = jnp.mean(h, axis=1, keepdims=True)
    var = jnp.mean((h - mu) ** 2, axis=1, keepdims=True)
    h = (h - mu) * lax.rsqrt(var + 1e-5) * lnw_ref[...][None, :] \
        + lnb_ref[...][None, :]
    ht_ref[...] = h.T


def _stage_e(acc2, den2, bias2, ln_w, ln_b):
    blk = 1024
    grid = NUM_NODES // blk
    return pl.pallas_call(
        _stage_e_body,
        grid=(grid,),
        in_specs=[
            pl.BlockSpec((NC, blk, HID), lambda i: (0, i, 0)),
            pl.BlockSpec((NC, blk, L), lambda i: (0, i, 0)),
            pl.BlockSpec((HID,), lambda i: (0,)),
            pl.BlockSpec((HID,), lambda i: (0,)),
            pl.BlockSpec((HID,), lambda i: (0,)),
        ],
        out_specs=pl.BlockSpec((HID, blk), lambda i: (0, i)),
        out_shape=jax.ShapeDtypeStruct((HID, NUM_NODES), jnp.float32),
    )(acc2, den2, bias2, ln_w, ln_b)


def _stage_f_body(ht_ref, wi1_ref, wh1_ref, bi1_ref, bh1_ref,
                  wi2_ref, wh2_ref, bi2_ref, bh2_ref, fcw_ref, fcb_ref,
                  out_ref):
    def gru_step(x, hprev, wi_ref, wh_ref, bi_ref, bh_ref):
        gi = jnp.dot(x, wi_ref[...], preferred_element_type=jnp.float32) \
            + bi_ref[...][None, :]
        gh = jnp.dot(hprev, wh_ref[...], preferred_element_type=jnp.float32) \
            + bh_ref[...][None, :]
        r = jax.nn.sigmoid(gi[:, :HID] + gh[:, :HID])
        z = jax.nn.sigmoid(gi[:, HID:2 * HID] + gh[:, HID:2 * HID])
        n = jnp.tanh(gi[:, 2 * HID:] + r * gh[:, 2 * HID:])
        return (1.0 - z) * n + z * hprev

    def step(t, carry):
        h1, h2 = carry
        xt = ht_ref[t]
        h1 = gru_step(xt, h1, wi1_ref, wh1_ref, bi1_ref, bh1_ref)
        h2 = gru_step(h1, h2, wi2_ref, wh2_ref, bi2_ref, bh2_ref)
        return (h1, h2)

    h0 = (jnp.zeros((B, HID), jnp.float32), jnp.zeros((B, HID), jnp.float32))
    h1, h2 = lax.fori_loop(0, HID, step, h0)
    out_ref[...] = jnp.dot(h2, fcw_ref[...],
                           preferred_element_type=jnp.float32) \
        + fcb_ref[...][None, :]


def _stage_f(ht3, Wi1T, Wh1T, bi1, bh1, Wi2T, Wh2T, bi2, bh2, fc_wT, fc_b):
    return pl.pallas_call(
        _stage_f_body,
        out_shape=jax.ShapeDtypeStruct((B, OUT), jnp.float32),
    )(ht3, Wi1T, Wh1T, bi1, bh1, Wi2T, Wh2T, bi2, bh2, fc_wT, fc_b)


def kernel(x, edge_index, W1, att_src1, att_dst1, bias1, W2, att_src2,
           att_dst2, bias2, ln_w, ln_b, Wi1, Wh1, bi1, bh1, Wi2, Wh2, bi2,
           bh2, fc_w, fc_b):
    x2d = x.reshape(NUM_NODES, T)
    src = edge_index[0]
    dst = edge_index[1]
    zrow = jnp.zeros((NUM_NODES, HID), jnp.float32)
    zden = jnp.zeros((NUM_NODES, L), jnp.float32)

    h1, a1 = _stage_a(x2d, W1, att_src1, att_dst1)

    _gat1_edges, _gat2_edges = _build_sc_kernels()

    acc1, den1 = _gat1_edges(h1, a1[0], a1[1], src, dst, zrow, zden)

    h2, a2 = _stage_c(acc1, den1, bias1, W2, att_src2, att_dst2)

    acc2, den2 = _gat2_edges(h2, a2[0], a2[1], src, dst, zrow, zden)

    ht = _stage_e(acc2, den2, bias2, ln_w, ln_b)
    ht3 = ht.reshape(HID, B, N)

    return _stage_f(ht3, Wi1.T, Wh1.T, bi1, bh1, Wi2.T, Wh2.T, bi2, bh2,
                    fc_w.T, fc_b)
